# baseline (speedup 1.0000x reference)
import functools

import jax
import jax.numpy as jnp
from jax import lax
from jax.experimental import pallas as pl
from jax.experimental.pallas import tpu as pltpu
from jax.experimental.pallas import tpu_sc as plsc

NN = 10000
EE = 320000
D = 128

NC = 2
NS = 16
NW = NC * NS

NP = 10240
RPT = NP // NS
K = 128
C = 80
CH = C // 2
EW_ = C * K
EP = NW * EW_
NCH = NW * C
DC = NCH // NS
DR = DC // CH

_DN = (((1,), (1,)), ((), ()))


def _dot(a, b):
  return lax.dot_general(a, b, _DN, precision=lax.Precision.HIGHEST,
                         preferred_element_type=jnp.float32)


def _deg_col(deg2):
  d = lax.dot_general(deg2, jnp.full((1, NC), 0.5, jnp.float32),
                      (((0,), (1,)), ((), ())),
                      precision=lax.Precision.HIGHEST,
                      preferred_element_type=jnp.float32) + 1.0
  return jnp.where(d > 0, lax.rsqrt(d), 0.0)


def _pre_body(ab_ref, x_ref, w_ref, av_ref, hl_ref, ew_ref):
  xv = x_ref[...]
  hl_ref[...] = jnp.concatenate(
      [_dot(xv, w_ref[...]), jnp.zeros((NP - NN, D), jnp.float32)], axis=0)
  s = jnp.sum(xv * av_ref[...], axis=1, keepdims=True) + ab_ref[0]
  ew_ref[...] = jnp.concatenate(
      [jnp.exp(s), jnp.zeros((NP - NN, 1), jnp.float32)], axis=0)


@jax.jit
def _tc_pre(x, w, av, ab):
  vm = pl.BlockSpec(memory_space=pltpu.MemorySpace.VMEM)
  return pl.pallas_call(
      _pre_body,
      in_specs=[pl.BlockSpec(memory_space=pltpu.MemorySpace.SMEM)] + [vm] * 3,
      out_specs=[vm, vm],
      out_shape=[
          jax.ShapeDtypeStruct((NP, D), jnp.float32),
          jax.ShapeDtypeStruct((NP, 1), jnp.float32),
      ],
  )(ab, x, w, av)


def _respre_body(ab_ref, deg_ref, sp_ref, hl_ref, b_ref, x_ref, w_ref, av_ref,
                 h1_ref, hl1_ref, ew1_ref):
  dinv = _deg_col(deg_ref[...])
  s_sum = sp_ref[0:NN] + sp_ref[NP:NP + NN]
  dc = dinv[0:NN]
  h_new = dc * s_sum + (dc * dc) * hl_ref[0:NN] + b_ref[...]
  h1 = jnp.maximum(h_new + x_ref[...], 0.0)
  h1_ref[...] = h1
  hl1_ref[...] = jnp.concatenate(
      [_dot(h1, w_ref[...]), jnp.zeros((NP - NN, D), jnp.float32)], axis=0)
  s = jnp.sum(h1 * av_ref[...], axis=1, keepdims=True) + ab_ref[0]
  ew1_ref[...] = jnp.concatenate(
      [jnp.exp(s), jnp.zeros((NP - NN, 1), jnp.float32)], axis=0)


@jax.jit
def _tc_respre(ab1, deg2, sp, hl, b, x, w1, av1):
  vm = pl.BlockSpec(memory_space=pltpu.MemorySpace.VMEM)
  return pl.pallas_call(
      _respre_body,
      compiler_params=pltpu.CompilerParams(
          vmem_limit_bytes=100 * 1024 * 1024),
      in_specs=[pl.BlockSpec(memory_space=pltpu.MemorySpace.SMEM)] + [vm] * 7,
      out_specs=[vm, vm, vm],
      out_shape=[
          jax.ShapeDtypeStruct((NN, D), jnp.float32),
          jax.ShapeDtypeStruct((NP, D), jnp.float32),
          jax.ShapeDtypeStruct((NP, 1), jnp.float32),
      ],
  )(ab1, deg2, sp, hl, b, x, w1, av1)


def _post1_body(deg_ref, sp_ref, hl_ref, b_ref, h1_ref, x_ref, out_ref):
  dinv = _deg_col(deg_ref[...])
  s_sum = sp_ref[0:NN] + sp_ref[NP:NP + NN]
  dc = dinv[0:NN]
  h_new = dc * s_sum + (dc * dc) * hl_ref[0:NN] + b_ref[...]
  out_ref[...] = h_new + h1_ref[...] + x_ref[...]


@jax.jit
def _tc_post1(deg2, sp, hl, b, h1, x):
  return pl.pallas_call(
      _post1_body,
      compiler_params=pltpu.CompilerParams(
          vmem_limit_bytes=100 * 1024 * 1024),
      out_shape=jax.ShapeDtypeStruct((NN, D), jnp.float32),
  )(deg2, sp, hl, b, h1, x)


def _layer_kernel_body(ew_hbm, hl_hbm, src2_hbm, dst2_hbm,
                       deg_out, g_out, s_out,
                       src_v, dst_v, rows0, rows1, vals0, vals1, dvec, evec,
                       ew_sh, deg_sh, acc_sh, sem0, sem1):
  cid = lax.axis_index("c")
  sid = lax.axis_index("s")
  wid = cid * NS + sid
  base = sid * RPT

  @pl.loop(0, K)
  def _zr(r):
    @pl.loop(0, D // 16)
    def _zc(c):
      rows0[r, pl.ds(c * 16, 16)] = jnp.zeros((16,), jnp.float32)

  for p in range(RPT // K):
    pltpu.sync_copy(rows0, acc_sh.at[pl.ds(base + p * K, K)])
    pltpu.sync_copy(rows0.at[0], deg_sh.at[pl.ds(base + p * K, K)])

  @pl.when(sid == 0)
  def _ld():
    pltpu.sync_copy(ew_hbm, ew_sh)

  plsc.subcore_barrier()

  for rnd in range(DR):
    cbase = sid * DC + rnd * CH
    pltpu.sync_copy(src2_hbm.at[pl.ds(cbase, CH)], src_v)
    pltpu.sync_copy(dst2_hbm.at[pl.ds(cbase, CH)], dst_v)

    pltpu.async_copy(ew_sh.at[src_v.at[0]], vals0, sem0)

    @pl.loop(0, CH // 2)
    def _dg(it):
      c0 = it * 2
      c1 = c0 + 1
      c2 = lax.rem(c0 + 2, CH)
      pltpu.async_copy(ew_sh.at[src_v.at[c1]], vals1, sem1)
      pltpu.make_async_copy(ew_hbm.at[pl.ds(0, K)], vals0, sem0).wait()
      pltpu.sync_copy(vals0, deg_sh.at[dst_v.at[c0]], add=True)
      pltpu.async_copy(ew_sh.at[src_v.at[c2]], vals0, sem0)
      pltpu.make_async_copy(ew_hbm.at[pl.ds(0, K)], vals1, sem1).wait()
      pltpu.sync_copy(vals1, deg_sh.at[dst_v.at[c1]], add=True)

    pltpu.make_async_copy(ew_hbm.at[pl.ds(0, K)], vals0, sem0).wait()

  plsc.subcore_barrier()

  pltpu.sync_copy(deg_sh.at[pl.ds(base, RPT)], dvec)
  pltpu.sync_copy(ew_sh.at[pl.ds(base, RPT)], evec)

  @pl.loop(0, RPT // 16)
  def _nw(i):
    x = dvec[pl.ds(i * 16, 16)] + 1.0
    bits = plsc.bitcast(x, jnp.int32)
    y = plsc.bitcast(jnp.int32(0x5F3759DF) - lax.shift_right_logical(bits, 1),
                     jnp.float32)
    y = y * (1.5 - 0.5 * x * y * y)
    y = y * (1.5 - 0.5 * x * y * y)
    y = y * (1.5 - 0.5 * x * y * y)
    dvec[pl.ds(i * 16, 16)] = y * evec[pl.ds(i * 16, 16)]

  for p in range(RPT // K):
    pltpu.sync_copy(hl_hbm.at[pl.ds(base + p * K, K)], rows0)

    @pl.loop(0, K // 16)
    def _sr(j):
      cfv = dvec[pl.ds(p * K + j * 16, 16)]
      for r in range(16):
        cf = cfv[r]
        row = j * 16 + r
        for c in range(D // 16):
          rows0[row, pl.ds(c * 16, 16)] = rows0[row, pl.ds(c * 16, 16)] * cf

    pltpu.sync_copy(rows0, g_out.at[pl.ds(cid * NP + base + p * K, K)])

  plsc.subcore_barrier()

  for half in range(2):
    pltpu.sync_copy(src2_hbm.at[pl.ds(wid * C + half * CH, CH)], src_v)
    pltpu.sync_copy(dst2_hbm.at[pl.ds(wid * C + half * CH, CH)], dst_v)

    @pl.loop(0, CH)
    def _ofr(r):
      @pl.loop(0, K // 16)
      def _ofc(c):
        src_v[r, pl.ds(c * 16, 16)] = (src_v[r, pl.ds(c * 16, 16)]
                                       + cid * NP)

    pltpu.async_copy(g_out.at[src_v.at[0]], rows0, sem0)

    @pl.loop(0, CH // 2)
    def _it(it):
      c0 = it * 2
      c1 = c0 + 1
      c2 = lax.rem(c0 + 2, CH)
      pltpu.async_copy(g_out.at[src_v.at[c1]], rows1, sem1)
      pltpu.make_async_copy(g_out.at[pl.ds(0, K)], rows0, sem0).wait()
      pltpu.sync_copy(rows0, acc_sh.at[dst_v.at[c0]], add=True)
      pltpu.async_copy(g_out.at[src_v.at[c2]], rows0, sem0)
      pltpu.make_async_copy(g_out.at[pl.ds(0, K)], rows1, sem1).wait()
      pltpu.sync_copy(rows1, acc_sh.at[dst_v.at[c1]], add=True)

    pltpu.make_async_copy(g_out.at[pl.ds(0, K)], rows0, sem0).wait()

  plsc.subcore_barrier()

  pltpu.sync_copy(acc_sh.at[pl.ds(base, RPT)],
                  s_out.at[pl.ds(cid * NP + base, RPT)])
  pltpu.sync_copy(deg_sh.at[pl.ds(base, RPT)],
                  deg_out.at[pl.ds(cid * NP + base, RPT)])


@functools.cache
def _sc_layer_call():
  mesh = plsc.VectorSubcoreMesh(core_axis_name="c", subcore_axis_name="s")
  return pl.kernel(
      _layer_kernel_body,
      out_type=[
          jax.ShapeDtypeStruct((NC * NP,), jnp.float32),
          jax.ShapeDtypeStruct((NC * NP, D), jnp.float32),
          jax.ShapeDtypeStruct((NC * NP, D), jnp.float32),
      ],
      mesh=mesh,
      compiler_params=pltpu.CompilerParams(needs_layout_passes=False),
      scratch_types=[
          pltpu.VMEM((CH, K), jnp.int32),
          pltpu.VMEM((CH, K), jnp.int32),
          pltpu.VMEM((K, D), jnp.float32),
          pltpu.VMEM((K, D), jnp.float32),
          pltpu.VMEM((K,), jnp.float32),
          pltpu.VMEM((K,), jnp.float32),
          pltpu.VMEM((RPT,), jnp.float32),
          pltpu.VMEM((RPT,), jnp.float32),
          pltpu.VMEM_SHARED((NP,), jnp.float32),
          pltpu.VMEM_SHARED((NP,), jnp.float32),
          pltpu.VMEM_SHARED((NP, D), jnp.float32),
          pltpu.SemaphoreType.DMA,
          pltpu.SemaphoreType.DMA,
      ],
  )


@jax.jit
def kernel(x, edge_index, A0, a0, W0, b0, A1, a1, W1, b1):
  src = edge_index[0].astype(jnp.int32)
  dst = edge_index[1].astype(jnp.int32)
  pad = NN + (jnp.arange(EP - EE, dtype=jnp.int32) % 16)
  src2 = jnp.concatenate([src, pad]).reshape(NCH, K)
  dst2 = jnp.concatenate([dst, pad]).reshape(NCH, K)

  av0 = A0.reshape(1, D)
  ab0 = a0.reshape(1)
  av1 = A1.reshape(1, D)
  ab1 = a1.reshape(1)
  b0r = b0.reshape(1, D)
  b1r = b1.reshape(1, D)

  layer_call = _sc_layer_call()

  hl0, ewp0 = _tc_pre(x, W0, av0, ab0)
  deg0, _, sp0 = layer_call(ewp0.reshape(NP), hl0, src2, dst2)
  h1, hl1, ewp1 = _tc_respre(ab1, deg0.reshape(NC, NP), sp0, hl0, b0r, x,
                             W1, av1)
  deg1, _, sp1 = layer_call(ewp1.reshape(NP), hl1, src2, dst2)
  out = _tc_post1(deg1.reshape(NC, NP), sp1, hl1, b1r, h1, x)
  return out


# speedup vs baseline: 1.1752x; 1.1752x over previous
import functools

import jax
import jax.numpy as jnp
from jax import lax
from jax.experimental import pallas as pl
from jax.experimental.pallas import tpu as pltpu
from jax.experimental.pallas import tpu_sc as plsc

NN = 10000
EE = 320000
D = 128

NC = 2
NS = 16
NW = NC * NS

NP = 10240
RPT = NP // NS
K = 128
C = 80
EW_ = C * K
EP = NW * EW_

_DN = (((1,), (1,)), ((), ()))


def _dot(a, b):
  return lax.dot_general(a, b, _DN, precision=lax.Precision.HIGHEST,
                         preferred_element_type=jnp.float32)


def _pre_body(ab_ref, x_ref, w_ref, av_ref, ei_ref,
              hl_ref, ew_ref, src2_ref, dst2_ref):
  xv = x_ref[...]
  hl_ref[...] = _dot(xv, w_ref[...])
  s = jnp.sum(xv * av_ref[...], axis=1, keepdims=True) + ab_ref[0]
  ew = jnp.exp(s)
  ew_ref[...] = jnp.concatenate(
      [ew, jnp.zeros((NP - NN, 1), jnp.float32)], axis=0)
  pad = NN + lax.rem(
      lax.broadcasted_iota(jnp.int32, (EP // K - EE // K, K), 1), 16)
  src2_ref[...] = jnp.concatenate([ei_ref[0], pad], axis=0)
  dst2_ref[...] = jnp.concatenate([ei_ref[1], pad], axis=0)


@jax.jit
def _tc_pre(x, w, av, ab, ei3):
  vm = pl.BlockSpec(memory_space=pltpu.MemorySpace.VMEM)
  return pl.pallas_call(
      _pre_body,
      in_specs=[pl.BlockSpec(memory_space=pltpu.MemorySpace.SMEM)] + [vm] * 4,
      out_specs=[vm, vm, vm, vm],
      out_shape=[
          jax.ShapeDtypeStruct((NN, D), jnp.float32),
          jax.ShapeDtypeStruct((NP, 1), jnp.float32),
          jax.ShapeDtypeStruct((EP // K, K), jnp.int32),
          jax.ShapeDtypeStruct((EP // K, K), jnp.int32),
      ],
  )(ab, x, w, av, ei3)


def _mid_body(degp_ref, ewp_ref, hl_ref, gp_ref, dinv_ref):
  d = lax.dot_general(
      degp_ref[...], jnp.ones((1, 2), jnp.float32),
      (((0,), (1,)), ((), ())),
      precision=lax.Precision.HIGHEST,
      preferred_element_type=jnp.float32) + 1.0
  dinv = jnp.where(d > 0, lax.rsqrt(d), 0.0)
  coef = dinv * ewp_ref[...]
  g = coef[0:NN] * hl_ref[...]
  gp_ref[...] = jnp.concatenate(
      [g, jnp.zeros((NP - NN, D), jnp.float32)], axis=0)
  dinv_ref[...] = dinv[0:NN]


@jax.jit
def _tc_mid(degp, ewp, hl):
  return pl.pallas_call(
      _mid_body,
      out_shape=[
          jax.ShapeDtypeStruct((NP, D), jnp.float32),
          jax.ShapeDtypeStruct((NN, 1), jnp.float32),
      ],
  )(degp, ewp, hl)


def _respre_body(ab_ref, sp_ref, dinv_ref, hl_ref, b_ref, x_ref, w_ref, av_ref,
                 h1_ref, hl1_ref, ew1_ref):
  s_sum = sp_ref[0:NN] + sp_ref[NP:NP + NN]
  dinv = dinv_ref[...]
  h_new = dinv * s_sum + (dinv * dinv) * hl_ref[...] + b_ref[...]
  h1 = jnp.maximum(h_new + x_ref[...], 0.0)
  h1_ref[...] = h1
  hl1_ref[...] = _dot(h1, w_ref[...])
  s = jnp.sum(h1 * av_ref[...], axis=1, keepdims=True) + ab_ref[0]
  ew1_ref[...] = jnp.concatenate(
      [jnp.exp(s), jnp.zeros((NP - NN, 1), jnp.float32)], axis=0)


@jax.jit
def _tc_respre(ab1, sp, dinv, hl, b, x, w1, av1):
  vm = pl.BlockSpec(memory_space=pltpu.MemorySpace.VMEM)
  return pl.pallas_call(
      _respre_body,
      compiler_params=pltpu.CompilerParams(
          vmem_limit_bytes=100 * 1024 * 1024),
      in_specs=[pl.BlockSpec(memory_space=pltpu.MemorySpace.SMEM)] + [vm] * 7,
      out_specs=[vm, vm, vm],
      out_shape=[
          jax.ShapeDtypeStruct((NN, D), jnp.float32),
          jax.ShapeDtypeStruct((NN, D), jnp.float32),
          jax.ShapeDtypeStruct((NP, 1), jnp.float32),
      ],
  )(ab1, sp, dinv, hl, b, x, w1, av1)


def _post1_body(sp_ref, dinv_ref, hl_ref, b_ref, h1_ref, x_ref, out_ref):
  s_sum = sp_ref[0:NN] + sp_ref[NP:NP + NN]
  dinv = dinv_ref[...]
  h_new = dinv * s_sum + (dinv * dinv) * hl_ref[...] + b_ref[...]
  out_ref[...] = h_new + h1_ref[...] + x_ref[...]


@jax.jit
def _tc_post1(sp, dinv, hl, b, h1, x):
  return pl.pallas_call(
      _post1_body,
      out_shape=jax.ShapeDtypeStruct((NN, D), jnp.float32),
  )(sp, dinv, hl, b, h1, x)


def _deg_kernel_body(ew_hbm, srcf_hbm, dst2_hbm, out_hbm,
                     ew_v, src_v, dst_v, vals_v, zb_v, deg_sh):
  cid = lax.axis_index("c")
  sid = lax.axis_index("s")
  wid = cid * NS + sid

  pltpu.sync_copy(ew_hbm, ew_v)
  pltpu.sync_copy(srcf_hbm.at[pl.ds(wid * EW_, EW_)], src_v)
  pltpu.sync_copy(dst2_hbm.at[pl.ds(wid * C, C)], dst_v)

  @pl.loop(0, RPT // 16)
  def _z(i):
    zb_v[pl.ds(i * 16, 16)] = jnp.zeros((16,), jnp.float32)

  pltpu.sync_copy(zb_v, deg_sh.at[pl.ds(sid * RPT, RPT)])
  plsc.subcore_barrier()

  @pl.loop(0, C)
  def _c(c):
    @pl.loop(0, K // 16)
    def _j(j):
      s16 = src_v[pl.ds(c * K + j * 16, 16)]
      vals_v[pl.ds(j * 16, 16)] = plsc.load_gather(ew_v, [s16])

    pltpu.sync_copy(vals_v, deg_sh.at[dst_v.at[c]], add=True)

  plsc.subcore_barrier()
  pltpu.sync_copy(deg_sh.at[pl.ds(sid * RPT, RPT)],
                  out_hbm.at[pl.ds(cid * NP + sid * RPT, RPT)])


@functools.cache
def _sc_deg_call():
  mesh = plsc.VectorSubcoreMesh(core_axis_name="c", subcore_axis_name="s")
  return pl.kernel(
      _deg_kernel_body,
      out_type=jax.ShapeDtypeStruct((NC * NP,), jnp.float32),
      mesh=mesh,
      compiler_params=pltpu.CompilerParams(needs_layout_passes=False),
      scratch_types=[
          pltpu.VMEM((NP,), jnp.float32),
          pltpu.VMEM((EW_,), jnp.int32),
          pltpu.VMEM((C, K), jnp.int32),
          pltpu.VMEM((K,), jnp.float32),
          pltpu.VMEM((RPT,), jnp.float32),
          pltpu.VMEM_SHARED((NP,), jnp.float32),
      ],
  )


CH = C // 2


def _spmm_kernel_body(g_hbm, src2_hbm, dst2_hbm, out_hbm,
                      src_v, dst_v, rows0, rows1, acc_sh, sem0, sem1):
  cid = lax.axis_index("c")
  sid = lax.axis_index("s")
  wid = cid * NS + sid

  @pl.loop(0, K)
  def _zr(r):
    @pl.loop(0, D // 16)
    def _zc(c):
      rows0[r, pl.ds(c * 16, 16)] = jnp.zeros((16,), jnp.float32)

  for p in range(RPT // K):
    pltpu.sync_copy(rows0, acc_sh.at[pl.ds(sid * RPT + p * K, K)])
  plsc.subcore_barrier()

  for half in range(2):
    pltpu.sync_copy(src2_hbm.at[pl.ds(wid * C + half * CH, CH)], src_v)
    pltpu.sync_copy(dst2_hbm.at[pl.ds(wid * C + half * CH, CH)], dst_v)

    pltpu.async_copy(g_hbm.at[src_v.at[0]], rows0, sem0)

    @pl.loop(0, CH // 2)
    def _it(it):
      c0 = it * 2
      c1 = c0 + 1
      c2 = lax.rem(c0 + 2, CH)
      pltpu.async_copy(g_hbm.at[src_v.at[c1]], rows1, sem1)
      pltpu.make_async_copy(g_hbm.at[pl.ds(0, K)], rows0, sem0).wait()
      pltpu.sync_copy(rows0, acc_sh.at[dst_v.at[c0]], add=True)
      pltpu.async_copy(g_hbm.at[src_v.at[c2]], rows0, sem0)
      pltpu.make_async_copy(g_hbm.at[pl.ds(0, K)], rows1, sem1).wait()
      pltpu.sync_copy(rows1, acc_sh.at[dst_v.at[c1]], add=True)

    pltpu.make_async_copy(g_hbm.at[pl.ds(0, K)], rows0, sem0).wait()

  plsc.subcore_barrier()
  pltpu.sync_copy(acc_sh.at[pl.ds(sid * RPT, RPT)],
                  out_hbm.at[pl.ds(cid * NP + sid * RPT, RPT)])


@functools.cache
def _sc_spmm_call():
  mesh = plsc.VectorSubcoreMesh(core_axis_name="c", subcore_axis_name="s")
  return pl.kernel(
      _spmm_kernel_body,
      out_type=jax.ShapeDtypeStruct((NC * NP, D), jnp.float32),
      mesh=mesh,
      compiler_params=pltpu.CompilerParams(needs_layout_passes=False),
      scratch_types=[
          pltpu.VMEM((CH, K), jnp.int32),
          pltpu.VMEM((CH, K), jnp.int32),
          pltpu.VMEM((K, D), jnp.float32),
          pltpu.VMEM((K, D), jnp.float32),
          pltpu.VMEM_SHARED((NP, D), jnp.float32),
          pltpu.SemaphoreType.DMA,
          pltpu.SemaphoreType.DMA,
      ],
  )


@jax.jit
def kernel(x, edge_index, A0, a0, W0, b0, A1, a1, W1, b1):
  ei3 = edge_index.astype(jnp.int32).reshape(2, EE // K, K)

  av0 = A0.reshape(1, D)
  ab0 = a0.reshape(1)
  av1 = A1.reshape(1, D)
  ab1 = a1.reshape(1)
  b0r = b0.reshape(1, D)
  b1r = b1.reshape(1, D)

  deg_call = _sc_deg_call()
  spmm_call = _sc_spmm_call()

  hl0, ewp0, src2, dst2 = _tc_pre(x, W0, av0, ab0, ei3)
  srcf = src2.reshape(EP)
  degp0 = deg_call(ewp0.reshape(NP), srcf, dst2).reshape(NC, NP)
  gp0, dinv0 = _tc_mid(degp0, ewp0, hl0)
  sp0 = spmm_call(gp0, src2, dst2)
  h1, hl1, ewp1 = _tc_respre(ab1, sp0, dinv0, hl0, b0r, x, W1, av1)
  degp1 = deg_call(ewp1.reshape(NP), srcf, dst2).reshape(NC, NP)
  gp1, dinv1 = _tc_mid(degp1, ewp1, hl1)
  sp1 = spmm_call(gp1, src2, dst2)
  out = _tc_post1(sp1, dinv1, hl1, b1r, h1, x)
  return out
